# Initial kernel scaffold; baseline (speedup 1.0000x reference)
import jax
import jax.numpy as jnp
from jax.experimental import pallas as pl


def kernel(noisy_types, noisy_pos, edge_index, t, params):
    raise NotImplementedError("write your pallas kernel here")


# speedup vs baseline: 1.8200x; 1.8200x over previous
import functools

import jax
import jax.numpy as jnp
from jax import lax
from jax.experimental import pallas as pl
from jax.experimental.pallas import tpu as pltpu
from jax.experimental.pallas import tpu_sc as plsc

N = 10000
E = 160000
H = 128
NT = 10
T = 500

NP = 10240
EP = 163840
NWORK = 32
CH = 128
EW = EP // NWORK
IW = EW // CH
RT = NP // 16
N8 = NP // 8

BE = 512
BN = 512

_f32 = jnp.float32
_i32 = jnp.int32


def _silu(x):
    return x * (1.0 / (1.0 + jnp.exp(-x)))


def _dot(a, b, preferred_element_type=None):
    return jnp.dot(a, b, preferred_element_type=preferred_element_type)


def _sc_gather(hs, hd, pos4, src2d, dst2d, z16):
    mesh = plsc.VectorSubcoreMesh(core_axis_name="c", subcore_axis_name="s")

    @functools.partial(
        pl.kernel,
        out_type=(
            jax.ShapeDtypeStruct((EP, H), _f32),
            jax.ShapeDtypeStruct((EP, H), _f32),
            jax.ShapeDtypeStruct((EP * 16,), _f32),
        ),
        mesh=mesh,
        compiler_params=pltpu.CompilerParams(needs_layout_passes=False),
        scratch_types=[
            pltpu.VMEM((IW, CH), _i32),
            pltpu.VMEM((IW, CH), _i32),
            pltpu.VMEM((NP * 4,), _f32),
            pltpu.VMEM((CH, H), _f32),
            pltpu.VMEM((CH, H), _f32),
            pltpu.VMEM((CH * 16,), _f32),
            pltpu.SemaphoreType.DMA,
            pltpu.SemaphoreType.DMA,
        ],
    )
    def gk(hs_r, hd_r, pos_r, src_r, dst_r, z16_r, hsg_o, hdg_o, dd_o,
           srcv, dstv, posv, bhs, bhd, bdd, gsem, wsem):
        cid = lax.axis_index("c")
        sid = lax.axis_index("s")
        wid = sid * 2 + cid
        pltpu.sync_copy(src_r.at[pl.ds(wid * IW, IW)], srcv)
        pltpu.sync_copy(dst_r.at[pl.ds(wid * IW, IW)], dstv)
        pltpu.sync_copy(pos_r, posv)
        pltpu.sync_copy(z16_r, bdd)
        base = wid * EW
        iota = lax.iota(_i32, 16)

        def body(c, carry):
            g1 = pltpu.async_copy(hs_r.at[srcv.at[c]], bhs, gsem)
            g2 = pltpu.async_copy(hd_r.at[dstv.at[c]], bhd, gsem)
            for g in range(8):
                rbase = (g * 16 + iota) * 16
                s4 = srcv[c, pl.ds(g * 16, 16)] * 4
                d4 = dstv[c, pl.ds(g * 16, 16)] * 4
                dx = plsc.load_gather(posv, [s4]) - \
                    plsc.load_gather(posv, [d4])
                dy = plsc.load_gather(posv, [s4 + 1]) - \
                    plsc.load_gather(posv, [d4 + 1])
                dz = plsc.load_gather(posv, [s4 + 2]) - \
                    plsc.load_gather(posv, [d4 + 2])
                dsq = dx * dx + dy * dy + dz * dz
                plsc.store_scatter(bdd, [rbase], dx)
                plsc.store_scatter(bdd, [rbase + 1], dy)
                plsc.store_scatter(bdd, [rbase + 2], dz)
                plsc.store_scatter(bdd, [rbase + 3], dsq)
            g1.wait()
            g2.wait()
            row = base + c * CH
            w1 = pltpu.async_copy(bhs, hsg_o.at[pl.ds(row, CH)], wsem)
            w2 = pltpu.async_copy(bhd, hdg_o.at[pl.ds(row, CH)], wsem)
            w3 = pltpu.async_copy(bdd, dd_o.at[pl.ds(row * 16, CH * 16)],
                                  wsem)
            w1.wait()
            w2.wait()
            w3.wait()
            return carry

        lax.fori_loop(0, IW, body, 0)

    return gk(hs, hd, pos4.reshape(NP * 4), src2d, dst2d,
              z16.reshape(CH * 16))


def _sc_scatter(m, cdx, dst2d, dst8_2d, zrt, with_coord):
    mesh = plsc.VectorSubcoreMesh(core_axis_name="c", subcore_axis_name="s")

    outs = [jax.ShapeDtypeStruct((NP, H), _f32),
            jax.ShapeDtypeStruct((NP, H), _f32)]
    scratch = [
        pltpu.VMEM((IW, CH), _i32),
        pltpu.VMEM((CH, H), _f32),
        pltpu.VMEM_SHARED((NP, H), _f32),
    ]
    if with_coord:
        outs += [jax.ShapeDtypeStruct((N8, H), _f32),
                 jax.ShapeDtypeStruct((N8, H), _f32)]
        scratch += [
            pltpu.VMEM((IW, CH), _i32),
            pltpu.VMEM_SHARED((N8, H), _f32),
        ]

    if with_coord:
        @functools.partial(pl.kernel, out_type=tuple(outs), mesh=mesh,
                           compiler_params=pltpu.CompilerParams(
                               needs_layout_passes=False),
                           scratch_types=scratch)
        def sk(m_r, cdx_r, dst_r, dst8_r, zrt_r,
               agg0_o, agg1_o, cp0_o, cp1_o,
               dstv, bm, aggs, dst8v, cds):
            cid = lax.axis_index("c")
            sid = lax.axis_index("s")
            wid = sid * 2 + cid
            pltpu.sync_copy(zrt_r, aggs.at[pl.ds(sid * RT, RT)])
            pltpu.sync_copy(zrt_r.at[pl.ds(0, N8 // 16)],
                            cds.at[pl.ds(sid * (N8 // 16), N8 // 16)])
            plsc.subcore_barrier()
            pltpu.sync_copy(dst_r.at[pl.ds(wid * IW, IW)], dstv)
            pltpu.sync_copy(dst8_r.at[pl.ds(wid * IW, IW)], dst8v)
            base = wid * EW

            def body(c, carry):
                row = base + c * CH
                pltpu.sync_copy(m_r.at[pl.ds(row, CH)], bm)
                pltpu.sync_copy(bm, aggs.at[dstv.at[c]], add=True)
                pltpu.sync_copy(cdx_r.at[pl.ds(row, CH)], bm)
                pltpu.sync_copy(bm, cds.at[dst8v.at[c]], add=True)
                return carry

            lax.fori_loop(0, IW, body, 0)
            plsc.subcore_barrier()
            rows = pl.ds(sid * RT, RT)
            crows = pl.ds(sid * (N8 // 16), N8 // 16)

            @pl.when(cid == 0)
            def _():
                pltpu.sync_copy(aggs.at[rows], agg0_o.at[rows])
                pltpu.sync_copy(cds.at[crows], cp0_o.at[crows])

            @pl.when(cid == 1)
            def _():
                pltpu.sync_copy(aggs.at[rows], agg1_o.at[rows])
                pltpu.sync_copy(cds.at[crows], cp1_o.at[crows])

        return sk(m, cdx, dst2d, dst8_2d, zrt)

    @functools.partial(pl.kernel, out_type=tuple(outs), mesh=mesh,
                       compiler_params=pltpu.CompilerParams(
                           needs_layout_passes=False),
                       scratch_types=scratch)
    def sk2(m_r, dst_r, zrt_r, agg0_o, agg1_o, dstv, bm, aggs):
        cid = lax.axis_index("c")
        sid = lax.axis_index("s")
        wid = sid * 2 + cid
        pltpu.sync_copy(zrt_r, aggs.at[pl.ds(sid * RT, RT)])
        plsc.subcore_barrier()
        pltpu.sync_copy(dst_r.at[pl.ds(wid * IW, IW)], dstv)
        base = wid * EW

        def body(c, carry):
            pltpu.sync_copy(m_r.at[pl.ds(base + c * CH, CH)], bm)
            pltpu.sync_copy(bm, aggs.at[dstv.at[c]], add=True)
            return carry

        lax.fori_loop(0, IW, body, 0)
        plsc.subcore_barrier()
        rows = pl.ds(sid * RT, RT)

        @pl.when(cid == 0)
        def _():
            pltpu.sync_copy(aggs.at[rows], agg0_o.at[rows])

        @pl.when(cid == 1)
        def _():
            pltpu.sync_copy(aggs.at[rows], agg1_o.at[rows])

    return sk2(m, dst2d, zrt)


def _row_spec():
    return pl.BlockSpec((1, H), lambda i: (0, 0))


def _wspec():
    return pl.BlockSpec((H, H), lambda i: (0, 0))


def _edge_tc(hsg, hdg, dd, dm8, w1t, mb1, mw2t, mb2,
             cw1t, cb1, cw2t, cb2, with_coord):
    grid = (EP // BE,)

    def msg_matmul(hsg_r, hdg_r, dd_r, w1_r, mb1_r, mw2_r, mb2_r):
        ddb = dd_r[...]
        dsq = ddb[:, 3:4]
        lane8 = lax.broadcasted_iota(_i32, (BE, 8), 1)
        dsq8 = jnp.where(lane8 == 0, dsq, 0.0)
        msg = jnp.concatenate([hsg_r[...], hdg_r[...], dsq8], axis=1)
        m1 = _silu(_dot(msg, w1_r[...], preferred_element_type=_f32)
                   + mb1_r[...])
        m = _silu(_dot(m1, mw2_r[...], preferred_element_type=_f32)
                  + mb2_r[...])
        return ddb, m

    def body_coord(hsg_r, hdg_r, dd_r, dm8_r, w1_r, mb1_r, mw2_r, mb2_r,
                   cw1_r, cb1_r, cw2_r, cb2_r, m_o, cdx_o):
        ddb, m = msg_matmul(hsg_r, hdg_r, dd_r, w1_r, mb1_r, mw2_r, mb2_r)
        m_o[...] = m
        u = _silu(_dot(m, cw1_r[...], preferred_element_type=_f32)
                  + cb1_r[...])
        cw8 = _dot(u, cw2_r[...], preferred_element_type=_f32)
        cwv = cw8[:, 0:1] + cb2_r[...]
        cwv = jnp.clip(cwv, -10.0, 10.0)
        cd16 = ddb * cwv
        tiled = jnp.concatenate([cd16] * 8, axis=1)
        lane = lax.broadcasted_iota(_i32, (BE, H), 1)
        keep = (lane // 16 == dm8_r[...]) & (lane % 16 < 3)
        cdx_o[...] = jnp.where(keep, tiled, 0.0)

    def body_plain(hsg_r, hdg_r, dd_r, w1_r, mb1_r, mw2_r, mb2_r, m_o):
        _, m = msg_matmul(hsg_r, hdg_r, dd_r, w1_r, mb1_r, mw2_r, mb2_r)
        m_o[...] = m

    e_spec = pl.BlockSpec((BE, H), lambda i: (i, 0))
    d_spec = pl.BlockSpec((BE, 16), lambda i: (i, 0))
    w1_spec = pl.BlockSpec((264, H), lambda i: (0, 0))
    if with_coord:
        in_specs = [e_spec, e_spec, d_spec,
                    pl.BlockSpec((BE, 1), lambda i: (i, 0)),
                    w1_spec, _row_spec(), _wspec(), _row_spec(),
                    _wspec(), _row_spec(),
                    pl.BlockSpec((H, 8), lambda i: (0, 0)),
                    pl.BlockSpec((1, 1), lambda i: (0, 0))]
        return pl.pallas_call(
            body_coord,
            grid=grid,
            in_specs=in_specs,
            out_specs=(e_spec, e_spec),
            out_shape=(jax.ShapeDtypeStruct((EP, H), _f32),
                       jax.ShapeDtypeStruct((EP, H), _f32)),
        )(hsg, hdg, dd, dm8, w1t, mb1, mw2t, mb2, cw1t, cb1, cw2t, cb2)
    in_specs = [e_spec, e_spec, d_spec,
                w1_spec, _row_spec(), _wspec(), _row_spec()]
    return pl.pallas_call(
        body_plain,
        grid=grid,
        in_specs=in_specs,
        out_specs=e_spec,
        out_shape=jax.ShapeDtypeStruct((EP, H), _f32),
    )(hsg, hdg, dd, w1t, mb1, mw2t, mb2)


def _node0_tc(nt144, w_in, b_in):
    grid = (NP // BN,)

    def body(nt_r, w_r, b_r, h_o):
        h_o[...] = _dot(nt_r[...], w_r[...], preferred_element_type=_f32) \
            + b_r[...]

    n_spec = pl.BlockSpec((BN, H), lambda i: (i, 0))
    return pl.pallas_call(
        body,
        grid=grid,
        in_specs=[pl.BlockSpec((BN, 144), lambda i: (i, 0)),
                  pl.BlockSpec((144, H), lambda i: (0, 0)),
                  _row_spec()],
        out_specs=n_spec,
        out_shape=jax.ShapeDtypeStruct((NP, H), _f32),
    )(nt144, w_in, b_in)


def _node_mid_tc(h, pos4, agg0, agg1, cda0, cda1, nw1t, nb1, nw2t, nb2):
    grid = (NP // BN,)

    def body(h_r, pos_r, a0_r, a1_r, c0_r, c1_r,
             w1_r, b1_r, w2_r, b2_r, h_o, pos_o):
        agg = a0_r[...] + a1_r[...]
        hn = jnp.concatenate([h_r[...], agg], axis=1)
        act = _silu(_dot(hn, w1_r[...], preferred_element_type=_f32)
                    + b1_r[...])
        h_o[...] = h_r[...] + _dot(act, w2_r[...],
                                   preferred_element_type=_f32) + b2_r[...]
        pos_o[...] = pos_r[...] + c0_r[...][:, :4] + c1_r[...][:, :4]

    n_spec = pl.BlockSpec((BN, H), lambda i: (i, 0))
    p_spec = pl.BlockSpec((BN, 4), lambda i: (i, 0))
    c_spec = pl.BlockSpec((BN, 16), lambda i: (i, 0))
    return pl.pallas_call(
        body,
        grid=grid,
        in_specs=[n_spec, p_spec, n_spec, n_spec, c_spec, c_spec,
                  pl.BlockSpec((2 * H, H), lambda i: (0, 0)), _row_spec(),
                  _wspec(), _row_spec()],
        out_specs=(n_spec, p_spec),
        out_shape=(jax.ShapeDtypeStruct((NP, H), _f32),
                   jax.ShapeDtypeStruct((NP, 4), _f32)),
    )(h, pos4, agg0, agg1, cda0, cda1, nw1t, nb1, nw2t, nb2)


def _node_fin_tc(h, agg0, agg1, nw1t, nb1, nw2t, nb2, cwt, cb, awt, ab):
    grid = (NP // BN,)

    def body(h_r, a0_r, a1_r, w1_r, b1_r, w2_r, b2_r,
             cw_r, cb_r, aw_r, ab_r, co_o, ao_o):
        agg = a0_r[...] + a1_r[...]
        hn = jnp.concatenate([h_r[...], agg], axis=1)
        act = _silu(_dot(hn, w1_r[...], preferred_element_type=_f32)
                    + b1_r[...])
        hf = h_r[...] + _dot(act, w2_r[...],
                             preferred_element_type=_f32) + b2_r[...]
        co_o[...] = _dot(hf, cw_r[...], preferred_element_type=_f32) \
            + cb_r[...]
        ao_o[...] = _dot(hf, aw_r[...], preferred_element_type=_f32) \
            + ab_r[...]

    n_spec = pl.BlockSpec((BN, H), lambda i: (i, 0))
    o_spec = pl.BlockSpec((BN, 16), lambda i: (i, 0))
    sw_spec = pl.BlockSpec((H, 16), lambda i: (0, 0))
    sr_spec = pl.BlockSpec((1, 16), lambda i: (0, 0))
    return pl.pallas_call(
        body,
        grid=grid,
        in_specs=[n_spec, n_spec, n_spec,
                  pl.BlockSpec((2 * H, H), lambda i: (0, 0)), _row_spec(),
                  _wspec(), _row_spec(),
                  sw_spec, sr_spec, sw_spec, sr_spec],
        out_specs=(o_spec, o_spec),
        out_shape=(jax.ShapeDtypeStruct((NP, 16), _f32),
                   jax.ShapeDtypeStruct((NP, 16), _f32)),
    )(h, agg0, agg1, nw1t, nb1, nw2t, nb2, cwt, cb, awt, ab)


def kernel(noisy_types, noisy_pos, edge_index, t, params):
    p = params
    tt = t.reshape(1, 1).astype(_f32) / T
    te = _silu(tt @ p['time_w1'].T + p['time_b1'])
    te = te @ p['time_w2'].T + p['time_b2']

    nt144 = jnp.zeros((NP, 144), _f32)
    nt144 = nt144.at[:N, :NT].set(noisy_types)
    nt144 = nt144.at[:, NT:NT + H].set(jnp.broadcast_to(te, (NP, H)))
    w_in = jnp.zeros((144, H), _f32).at[:NT + H, :].set(p['in_w'].T)
    b_in = p['in_b'].reshape(1, H)

    pos = jnp.zeros((NP, 4), _f32).at[:N, :3].set(noisy_pos)

    src = edge_index[0].astype(_i32)
    dst = edge_index[1].astype(_i32)
    src2d = jnp.zeros((EP,), _i32).at[:E].set(src).reshape(EP // CH, CH)
    dst_p = jnp.full((EP,), N, _i32).at[:E].set(dst)
    dst2d = dst_p.reshape(EP // CH, CH)
    dst8_2d = (dst_p >> 3).reshape(EP // CH, CH)
    dm8 = (dst_p & 7).reshape(EP, 1)

    zrt = jnp.zeros((RT, H), _f32)
    z16 = jnp.zeros((CH, 16), _f32)

    layers = p['layers']
    h = _node0_tc(nt144, w_in, b_in)

    for li in range(4):
        lp = layers[li]
        w1t = jnp.zeros((264, H), _f32).at[:2 * H + 1, :].set(lp['mw1'].T)
        mb1 = lp['mb1'].reshape(1, H)
        mw2t = lp['mw2'].T
        mb2 = lp['mb2'].reshape(1, H)
        nw1t = lp['nw1'].T
        nb1 = lp['nb1'].reshape(1, H)
        nw2t = lp['nw2'].T
        nb2 = lp['nb2'].reshape(1, H)

        hsg, hdg, ddf = _sc_gather(h, h, pos, src2d, dst2d, z16)
        dd = ddf.reshape(EP, 16)

        if li < 3:
            cw1t = lp['cw1'].T
            cb1 = lp['cb1'].reshape(1, H)
            cw2t = jnp.zeros((H, 8), _f32).at[:, 0].set(lp['cw2'][0])
            cb2 = lp['cb2'].reshape(1, 1)
            m, cdx = _edge_tc(hsg, hdg, dd, dm8, w1t, mb1, mw2t, mb2,
                              cw1t, cb1, cw2t, cb2, with_coord=True)
            agg0, agg1, cp0, cp1 = _sc_scatter(m, cdx, dst2d, dst8_2d, zrt,
                                               with_coord=True)
            cda0 = cp0.reshape(NP, 16)
            cda1 = cp1.reshape(NP, 16)
            h, pos = _node_mid_tc(h, pos, agg0, agg1, cda0, cda1,
                                  nw1t, nb1, nw2t, nb2)
        else:
            m = _edge_tc(hsg, hdg, dd, None, w1t, mb1, mw2t, mb2,
                         None, None, None, None, with_coord=False)
            agg0, agg1 = _sc_scatter(m, None, dst2d, None, zrt,
                                     with_coord=False)
            cwt = jnp.zeros((H, 16), _f32).at[:, :3].set(p['coord_w'].T)
            cb = jnp.zeros((1, 16), _f32).at[0, :3].set(p['coord_b'])
            awt = jnp.zeros((H, 16), _f32).at[:, :NT].set(p['atom_w'].T)
            ab = jnp.zeros((1, 16), _f32).at[0, :NT].set(p['atom_b'])
            co, ao = _node_fin_tc(h, agg0, agg1, nw1t, nb1,
                                  nw2t, nb2, cwt, cb, awt, ab)

    return (co[:N, :3], ao[:N, :NT])


# speedup vs baseline: 1.9968x; 1.0972x over previous
import functools

import jax
import jax.numpy as jnp
from jax import lax
from jax.experimental import pallas as pl
from jax.experimental.pallas import tpu as pltpu
from jax.experimental.pallas import tpu_sc as plsc

N = 10000
E = 160000
H = 128
NT = 10
T = 500

NP = 10240
EP = 163840
NWORK = 32
CH = 128
EW = EP // NWORK
IW = EW // CH
RT = NP // 16
N8 = NP // 8

BE = 512
BN = 512

_f32 = jnp.float32
_i32 = jnp.int32


def _silu(x):
    return x * (1.0 / (1.0 + jnp.exp(-x)))


def _dot(a, b, preferred_element_type=None):
    return jnp.dot(a, b, preferred_element_type=preferred_element_type)


def _sc_gather(hs, hd, pos4, src2d, dst2d, z16):
    mesh = plsc.VectorSubcoreMesh(core_axis_name="c", subcore_axis_name="s")

    @functools.partial(
        pl.kernel,
        out_type=(
            jax.ShapeDtypeStruct((EP, H), _f32),
            jax.ShapeDtypeStruct((EP, H), _f32),
            jax.ShapeDtypeStruct((EP * 16,), _f32),
        ),
        mesh=mesh,
        compiler_params=pltpu.CompilerParams(needs_layout_passes=False),
        scratch_types=[
            pltpu.VMEM((IW, CH), _i32),
            pltpu.VMEM((IW, CH), _i32),
            pltpu.VMEM((NP * 4,), _f32),
            pltpu.VMEM((CH, H), _f32),
            pltpu.VMEM((CH, H), _f32),
            pltpu.VMEM((CH * 16,), _f32),
            pltpu.VMEM((CH, H), _f32),
            pltpu.VMEM((CH, H), _f32),
            pltpu.VMEM((CH * 16,), _f32),
            pltpu.SemaphoreType.DMA,
            pltpu.SemaphoreType.DMA,
            pltpu.SemaphoreType.DMA,
        ],
    )
    def gk(hs_r, hd_r, pos_r, src_r, dst_r, z16_r, hsg_o, hdg_o, dd_o,
           srcv, dstv, posv, bhsA, bhdA, bddA, bhsB, bhdB, bddB,
           gsemA, gsemB, wsem):
        cid = lax.axis_index("c")
        sid = lax.axis_index("s")
        wid = sid * 2 + cid
        pltpu.sync_copy(src_r.at[pl.ds(wid * IW, IW)], srcv)
        pltpu.sync_copy(dst_r.at[pl.ds(wid * IW, IW)], dstv)
        pltpu.sync_copy(pos_r, posv)
        pltpu.sync_copy(z16_r, bddA)
        pltpu.sync_copy(z16_r, bddB)
        base = wid * EW
        iota = lax.iota(_i32, 16)

        def start_gather(c, bhs, bhd, gsem):
            g1 = pltpu.async_copy(hs_r.at[srcv.at[c]], bhs, gsem)
            g2 = pltpu.async_copy(hd_r.at[dstv.at[c]], bhd, gsem)
            return g1, g2

        def compute_dd(c, bdd):
            for g in range(8):
                rbase = (g * 16 + iota) * 16
                s4 = srcv[c, pl.ds(g * 16, 16)] * 4
                d4 = dstv[c, pl.ds(g * 16, 16)] * 4
                dx = plsc.load_gather(posv, [s4]) - \
                    plsc.load_gather(posv, [d4])
                dy = plsc.load_gather(posv, [s4 + 1]) - \
                    plsc.load_gather(posv, [d4 + 1])
                dz = plsc.load_gather(posv, [s4 + 2]) - \
                    plsc.load_gather(posv, [d4 + 2])
                dsq = dx * dx + dy * dy + dz * dz
                plsc.store_scatter(bdd, [rbase], dx)
                plsc.store_scatter(bdd, [rbase + 1], dy)
                plsc.store_scatter(bdd, [rbase + 2], dz)
                plsc.store_scatter(bdd, [rbase + 3], dsq)

        def write_out(c, bhs, bhd, bdd):
            row = base + c * CH
            w1 = pltpu.async_copy(bhs, hsg_o.at[pl.ds(row, CH)], wsem)
            w2 = pltpu.async_copy(bhd, hdg_o.at[pl.ds(row, CH)], wsem)
            w3 = pltpu.async_copy(bdd, dd_o.at[pl.ds(row * 16, CH * 16)],
                                  wsem)
            w1.wait()
            w2.wait()
            w3.wait()

        start_gather(0, bhsA, bhdA, gsemA)

        def body2(k, carry):
            c0 = 2 * k
            c1 = c0 + 1
            start_gather(c1, bhsB, bhdB, gsemB)
            pltpu.make_async_copy(hs_r.at[srcv.at[c0]], bhsA, gsemA).wait()
            pltpu.make_async_copy(hd_r.at[dstv.at[c0]], bhdA, gsemA).wait()
            compute_dd(c0, bddA)
            write_out(c0, bhsA, bhdA, bddA)

            @pl.when(k < IW // 2 - 1)
            def _():
                start_gather(c0 + 2, bhsA, bhdA, gsemA)

            pltpu.make_async_copy(hs_r.at[srcv.at[c1]], bhsB, gsemB).wait()
            pltpu.make_async_copy(hd_r.at[dstv.at[c1]], bhdB, gsemB).wait()
            compute_dd(c1, bddB)
            write_out(c1, bhsB, bhdB, bddB)
            return carry

        lax.fori_loop(0, IW // 2, body2, 0)

    return gk(hs, hd, pos4.reshape(NP * 4), src2d, dst2d,
              z16.reshape(CH * 16))


def _sc_scatter(m, cdx, dst2d, dst8_2d, zrt, with_coord):
    mesh = plsc.VectorSubcoreMesh(core_axis_name="c", subcore_axis_name="s")

    outs = [jax.ShapeDtypeStruct((NP, H), _f32),
            jax.ShapeDtypeStruct((NP, H), _f32)]
    scratch = [
        pltpu.VMEM((IW, CH), _i32),
        pltpu.VMEM((CH, H), _f32),
        pltpu.VMEM_SHARED((NP, H), _f32),
    ]
    if with_coord:
        outs += [jax.ShapeDtypeStruct((N8, H), _f32),
                 jax.ShapeDtypeStruct((N8, H), _f32)]
        scratch += [
            pltpu.VMEM((IW, CH), _i32),
            pltpu.VMEM_SHARED((N8, H), _f32),
        ]

    if with_coord:
        @functools.partial(pl.kernel, out_type=tuple(outs), mesh=mesh,
                           compiler_params=pltpu.CompilerParams(
                               needs_layout_passes=False),
                           scratch_types=scratch)
        def sk(m_r, cdx_r, dst_r, dst8_r, zrt_r,
               agg0_o, agg1_o, cp0_o, cp1_o,
               dstv, bm, aggs, dst8v, cds):
            cid = lax.axis_index("c")
            sid = lax.axis_index("s")
            wid = sid * 2 + cid
            pltpu.sync_copy(zrt_r, aggs.at[pl.ds(sid * RT, RT)])
            pltpu.sync_copy(zrt_r.at[pl.ds(0, N8 // 16)],
                            cds.at[pl.ds(sid * (N8 // 16), N8 // 16)])
            plsc.subcore_barrier()
            pltpu.sync_copy(dst_r.at[pl.ds(wid * IW, IW)], dstv)
            pltpu.sync_copy(dst8_r.at[pl.ds(wid * IW, IW)], dst8v)
            base = wid * EW

            def body(c, carry):
                row = base + c * CH
                pltpu.sync_copy(m_r.at[pl.ds(row, CH)], bm)
                pltpu.sync_copy(bm, aggs.at[dstv.at[c]], add=True)
                pltpu.sync_copy(cdx_r.at[pl.ds(row, CH)], bm)
                pltpu.sync_copy(bm, cds.at[dst8v.at[c]], add=True)
                return carry

            lax.fori_loop(0, IW, body, 0)
            plsc.subcore_barrier()
            rows = pl.ds(sid * RT, RT)
            crows = pl.ds(sid * (N8 // 16), N8 // 16)

            @pl.when(cid == 0)
            def _():
                pltpu.sync_copy(aggs.at[rows], agg0_o.at[rows])
                pltpu.sync_copy(cds.at[crows], cp0_o.at[crows])

            @pl.when(cid == 1)
            def _():
                pltpu.sync_copy(aggs.at[rows], agg1_o.at[rows])
                pltpu.sync_copy(cds.at[crows], cp1_o.at[crows])

        return sk(m, cdx, dst2d, dst8_2d, zrt)

    @functools.partial(pl.kernel, out_type=tuple(outs), mesh=mesh,
                       compiler_params=pltpu.CompilerParams(
                           needs_layout_passes=False),
                       scratch_types=scratch)
    def sk2(m_r, dst_r, zrt_r, agg0_o, agg1_o, dstv, bm, aggs):
        cid = lax.axis_index("c")
        sid = lax.axis_index("s")
        wid = sid * 2 + cid
        pltpu.sync_copy(zrt_r, aggs.at[pl.ds(sid * RT, RT)])
        plsc.subcore_barrier()
        pltpu.sync_copy(dst_r.at[pl.ds(wid * IW, IW)], dstv)
        base = wid * EW

        def body(c, carry):
            pltpu.sync_copy(m_r.at[pl.ds(base + c * CH, CH)], bm)
            pltpu.sync_copy(bm, aggs.at[dstv.at[c]], add=True)
            return carry

        lax.fori_loop(0, IW, body, 0)
        plsc.subcore_barrier()
        rows = pl.ds(sid * RT, RT)

        @pl.when(cid == 0)
        def _():
            pltpu.sync_copy(aggs.at[rows], agg0_o.at[rows])

        @pl.when(cid == 1)
        def _():
            pltpu.sync_copy(aggs.at[rows], agg1_o.at[rows])

    return sk2(m, dst2d, zrt)


def _row_spec():
    return pl.BlockSpec((1, H), lambda i: (0, 0))


def _wspec():
    return pl.BlockSpec((H, H), lambda i: (0, 0))


def _edge_tc(hsg, hdg, dd, dm8, w1t, mb1, mw2t, mb2,
             cw1t, cb1, cw2t, cb2, with_coord):
    grid = (EP // BE,)

    def msg_matmul(hsg_r, hdg_r, dd_r, w1_r, mb1_r, mw2_r, mb2_r):
        ddb = dd_r[...]
        dsq = ddb[:, 3:4]
        lane8 = lax.broadcasted_iota(_i32, (BE, 8), 1)
        dsq8 = jnp.where(lane8 == 0, dsq, 0.0)
        msg = jnp.concatenate([hsg_r[...], hdg_r[...], dsq8], axis=1)
        m1 = _silu(_dot(msg, w1_r[...], preferred_element_type=_f32)
                   + mb1_r[...])
        m = _silu(_dot(m1, mw2_r[...], preferred_element_type=_f32)
                  + mb2_r[...])
        return ddb, m

    def body_coord(hsg_r, hdg_r, dd_r, dm8_r, w1_r, mb1_r, mw2_r, mb2_r,
                   cw1_r, cb1_r, cw2_r, cb2_r, m_o, cdx_o):
        ddb, m = msg_matmul(hsg_r, hdg_r, dd_r, w1_r, mb1_r, mw2_r, mb2_r)
        m_o[...] = m
        u = _silu(_dot(m, cw1_r[...], preferred_element_type=_f32)
                  + cb1_r[...])
        cw8 = _dot(u, cw2_r[...], preferred_element_type=_f32)
        cwv = cw8[:, 0:1] + cb2_r[...]
        cwv = jnp.clip(cwv, -10.0, 10.0)
        cd16 = ddb * cwv
        tiled = jnp.concatenate([cd16] * 8, axis=1)
        lane = lax.broadcasted_iota(_i32, (BE, H), 1)
        keep = (lane // 16 == dm8_r[...]) & (lane % 16 < 3)
        cdx_o[...] = jnp.where(keep, tiled, 0.0)

    def body_plain(hsg_r, hdg_r, dd_r, w1_r, mb1_r, mw2_r, mb2_r, m_o):
        _, m = msg_matmul(hsg_r, hdg_r, dd_r, w1_r, mb1_r, mw2_r, mb2_r)
        m_o[...] = m

    e_spec = pl.BlockSpec((BE, H), lambda i: (i, 0))
    d_spec = pl.BlockSpec((BE, 16), lambda i: (i, 0))
    w1_spec = pl.BlockSpec((264, H), lambda i: (0, 0))
    if with_coord:
        in_specs = [e_spec, e_spec, d_spec,
                    pl.BlockSpec((BE, 1), lambda i: (i, 0)),
                    w1_spec, _row_spec(), _wspec(), _row_spec(),
                    _wspec(), _row_spec(),
                    pl.BlockSpec((H, 8), lambda i: (0, 0)),
                    pl.BlockSpec((1, 1), lambda i: (0, 0))]
        return pl.pallas_call(
            body_coord,
            grid=grid,
            in_specs=in_specs,
            out_specs=(e_spec, e_spec),
            out_shape=(jax.ShapeDtypeStruct((EP, H), _f32),
                       jax.ShapeDtypeStruct((EP, H), _f32)),
        )(hsg, hdg, dd, dm8, w1t, mb1, mw2t, mb2, cw1t, cb1, cw2t, cb2)
    in_specs = [e_spec, e_spec, d_spec,
                w1_spec, _row_spec(), _wspec(), _row_spec()]
    return pl.pallas_call(
        body_plain,
        grid=grid,
        in_specs=in_specs,
        out_specs=e_spec,
        out_shape=jax.ShapeDtypeStruct((EP, H), _f32),
    )(hsg, hdg, dd, w1t, mb1, mw2t, mb2)


def _node0_tc(nt144, w_in, b_in):
    grid = (NP // BN,)

    def body(nt_r, w_r, b_r, h_o):
        h_o[...] = _dot(nt_r[...], w_r[...], preferred_element_type=_f32) \
            + b_r[...]

    n_spec = pl.BlockSpec((BN, H), lambda i: (i, 0))
    return pl.pallas_call(
        body,
        grid=grid,
        in_specs=[pl.BlockSpec((BN, 144), lambda i: (i, 0)),
                  pl.BlockSpec((144, H), lambda i: (0, 0)),
                  _row_spec()],
        out_specs=n_spec,
        out_shape=jax.ShapeDtypeStruct((NP, H), _f32),
    )(nt144, w_in, b_in)


def _node_mid_tc(h, pos4, agg0, agg1, cda0, cda1, nw1t, nb1, nw2t, nb2):
    grid = (NP // BN,)

    def body(h_r, pos_r, a0_r, a1_r, c0_r, c1_r,
             w1_r, b1_r, w2_r, b2_r, h_o, pos_o):
        agg = a0_r[...] + a1_r[...]
        hn = jnp.concatenate([h_r[...], agg], axis=1)
        act = _silu(_dot(hn, w1_r[...], preferred_element_type=_f32)
                    + b1_r[...])
        h_o[...] = h_r[...] + _dot(act, w2_r[...],
                                   preferred_element_type=_f32) + b2_r[...]
        pos_o[...] = pos_r[...] + c0_r[...][:, :4] + c1_r[...][:, :4]

    n_spec = pl.BlockSpec((BN, H), lambda i: (i, 0))
    p_spec = pl.BlockSpec((BN, 4), lambda i: (i, 0))
    c_spec = pl.BlockSpec((BN, 16), lambda i: (i, 0))
    return pl.pallas_call(
        body,
        grid=grid,
        in_specs=[n_spec, p_spec, n_spec, n_spec, c_spec, c_spec,
                  pl.BlockSpec((2 * H, H), lambda i: (0, 0)), _row_spec(),
                  _wspec(), _row_spec()],
        out_specs=(n_spec, p_spec),
        out_shape=(jax.ShapeDtypeStruct((NP, H), _f32),
                   jax.ShapeDtypeStruct((NP, 4), _f32)),
    )(h, pos4, agg0, agg1, cda0, cda1, nw1t, nb1, nw2t, nb2)


def _node_fin_tc(h, agg0, agg1, nw1t, nb1, nw2t, nb2, cwt, cb, awt, ab):
    grid = (NP // BN,)

    def body(h_r, a0_r, a1_r, w1_r, b1_r, w2_r, b2_r,
             cw_r, cb_r, aw_r, ab_r, co_o, ao_o):
        agg = a0_r[...] + a1_r[...]
        hn = jnp.concatenate([h_r[...], agg], axis=1)
        act = _silu(_dot(hn, w1_r[...], preferred_element_type=_f32)
                    + b1_r[...])
        hf = h_r[...] + _dot(act, w2_r[...],
                             preferred_element_type=_f32) + b2_r[...]
        co_o[...] = _dot(hf, cw_r[...], preferred_element_type=_f32) \
            + cb_r[...]
        ao_o[...] = _dot(hf, aw_r[...], preferred_element_type=_f32) \
            + ab_r[...]

    n_spec = pl.BlockSpec((BN, H), lambda i: (i, 0))
    o_spec = pl.BlockSpec((BN, 16), lambda i: (i, 0))
    sw_spec = pl.BlockSpec((H, 16), lambda i: (0, 0))
    sr_spec = pl.BlockSpec((1, 16), lambda i: (0, 0))
    return pl.pallas_call(
        body,
        grid=grid,
        in_specs=[n_spec, n_spec, n_spec,
                  pl.BlockSpec((2 * H, H), lambda i: (0, 0)), _row_spec(),
                  _wspec(), _row_spec(),
                  sw_spec, sr_spec, sw_spec, sr_spec],
        out_specs=(o_spec, o_spec),
        out_shape=(jax.ShapeDtypeStruct((NP, 16), _f32),
                   jax.ShapeDtypeStruct((NP, 16), _f32)),
    )(h, agg0, agg1, nw1t, nb1, nw2t, nb2, cwt, cb, awt, ab)


def kernel(noisy_types, noisy_pos, edge_index, t, params):
    p = params
    tt = t.reshape(1, 1).astype(_f32) / T
    te = _silu(tt @ p['time_w1'].T + p['time_b1'])
    te = te @ p['time_w2'].T + p['time_b2']

    nt144 = jnp.zeros((NP, 144), _f32)
    nt144 = nt144.at[:N, :NT].set(noisy_types)
    nt144 = nt144.at[:, NT:NT + H].set(jnp.broadcast_to(te, (NP, H)))
    w_in = jnp.zeros((144, H), _f32).at[:NT + H, :].set(p['in_w'].T)
    b_in = p['in_b'].reshape(1, H)

    pos = jnp.zeros((NP, 4), _f32).at[:N, :3].set(noisy_pos)

    src = edge_index[0].astype(_i32)
    dst = edge_index[1].astype(_i32)
    src2d = jnp.zeros((EP,), _i32).at[:E].set(src).reshape(EP // CH, CH)
    dst_p = jnp.full((EP,), N, _i32).at[:E].set(dst)
    dst2d = dst_p.reshape(EP // CH, CH)
    dst8_2d = (dst_p >> 3).reshape(EP // CH, CH)
    dm8 = (dst_p & 7).reshape(EP, 1)

    zrt = jnp.zeros((RT, H), _f32)
    z16 = jnp.zeros((CH, 16), _f32)

    layers = p['layers']
    h = _node0_tc(nt144, w_in, b_in)

    for li in range(4):
        lp = layers[li]
        w1t = jnp.zeros((264, H), _f32).at[:2 * H + 1, :].set(lp['mw1'].T)
        mb1 = lp['mb1'].reshape(1, H)
        mw2t = lp['mw2'].T
        mb2 = lp['mb2'].reshape(1, H)
        nw1t = lp['nw1'].T
        nb1 = lp['nb1'].reshape(1, H)
        nw2t = lp['nw2'].T
        nb2 = lp['nb2'].reshape(1, H)

        hsg, hdg, ddf = _sc_gather(h, h, pos, src2d, dst2d, z16)
        dd = ddf.reshape(EP, 16)

        if li < 3:
            cw1t = lp['cw1'].T
            cb1 = lp['cb1'].reshape(1, H)
            cw2t = jnp.zeros((H, 8), _f32).at[:, 0].set(lp['cw2'][0])
            cb2 = lp['cb2'].reshape(1, 1)
            m, cdx = _edge_tc(hsg, hdg, dd, dm8, w1t, mb1, mw2t, mb2,
                              cw1t, cb1, cw2t, cb2, with_coord=True)
            agg0, agg1, cp0, cp1 = _sc_scatter(m, cdx, dst2d, dst8_2d, zrt,
                                               with_coord=True)
            cda0 = cp0.reshape(NP, 16)
            cda1 = cp1.reshape(NP, 16)
            h, pos = _node_mid_tc(h, pos, agg0, agg1, cda0, cda1,
                                  nw1t, nb1, nw2t, nb2)
        else:
            m = _edge_tc(hsg, hdg, dd, None, w1t, mb1, mw2t, mb2,
                         None, None, None, None, with_coord=False)
            agg0, agg1 = _sc_scatter(m, None, dst2d, None, zrt,
                                     with_coord=False)
            cwt = jnp.zeros((H, 16), _f32).at[:, :3].set(p['coord_w'].T)
            cb = jnp.zeros((1, 16), _f32).at[0, :3].set(p['coord_b'])
            awt = jnp.zeros((H, 16), _f32).at[:, :NT].set(p['atom_w'].T)
            ab = jnp.zeros((1, 16), _f32).at[0, :NT].set(p['atom_b'])
            co, ao = _node_fin_tc(h, agg0, agg1, nw1t, nb1,
                                  nw2t, nb2, cwt, cb, awt, ab)

    return (co[:N, :3], ao[:N, :NT])


# speedup vs baseline: 2.0472x; 1.0253x over previous
import functools

import jax
import jax.numpy as jnp
from jax import lax
from jax.experimental import pallas as pl
from jax.experimental.pallas import tpu as pltpu
from jax.experimental.pallas import tpu_sc as plsc

N = 10000
E = 160000
H = 128
NT = 10
T = 500

NP = 10240
EP = 163840
NWORK = 32
CH = 128
EW = EP // NWORK
IW = EW // CH
RT = NP // 16
N8 = NP // 8

BE = 512
BN = 512

_f32 = jnp.float32
_i32 = jnp.int32


def _silu(x):
    return x * (1.0 / (1.0 + jnp.exp(-x)))


def _dot(a, b, preferred_element_type=None):
    return jnp.dot(a, b, preferred_element_type=preferred_element_type)


def _sc_gather(hs, hd, pos4, src2d, dst2d, z16):
    mesh = plsc.VectorSubcoreMesh(core_axis_name="c", subcore_axis_name="s")

    @functools.partial(
        pl.kernel,
        out_type=(
            jax.ShapeDtypeStruct((EP, H), _f32),
            jax.ShapeDtypeStruct((EP, H), _f32),
            jax.ShapeDtypeStruct((EP * 16,), _f32),
        ),
        mesh=mesh,
        compiler_params=pltpu.CompilerParams(needs_layout_passes=False),
        scratch_types=[
            pltpu.VMEM((IW, CH), _i32),
            pltpu.VMEM((IW, CH), _i32),
            pltpu.VMEM((NP * 4,), _f32),
            pltpu.VMEM((CH, H), _f32),
            pltpu.VMEM((CH, H), _f32),
            pltpu.VMEM((CH * 16,), _f32),
            pltpu.VMEM((CH, H), _f32),
            pltpu.VMEM((CH, H), _f32),
            pltpu.VMEM((CH * 16,), _f32),
            pltpu.SemaphoreType.DMA,
            pltpu.SemaphoreType.DMA,
            pltpu.SemaphoreType.DMA,
        ],
    )
    def gk(hs_r, hd_r, pos_r, src_r, dst_r, z16_r, hsg_o, hdg_o, dd_o,
           srcv, dstv, posv, bhsA, bhdA, bddA, bhsB, bhdB, bddB,
           gsemA, gsemB, wsem):
        cid = lax.axis_index("c")
        sid = lax.axis_index("s")
        wid = sid * 2 + cid
        pltpu.sync_copy(src_r.at[pl.ds(wid * IW, IW)], srcv)
        pltpu.sync_copy(dst_r.at[pl.ds(wid * IW, IW)], dstv)
        pltpu.sync_copy(pos_r, posv)
        pltpu.sync_copy(z16_r, bddA)
        pltpu.sync_copy(z16_r, bddB)
        base = wid * EW
        iota = lax.iota(_i32, 16)

        def start_gather(c, bhs, bhd, gsem):
            g1 = pltpu.async_copy(hs_r.at[srcv.at[c]], bhs, gsem)
            g2 = pltpu.async_copy(hd_r.at[dstv.at[c]], bhd, gsem)
            return g1, g2

        def compute_dd(c, bdd):
            for g in range(8):
                rbase = (g * 16 + iota) * 16
                s4 = srcv[c, pl.ds(g * 16, 16)] * 4
                d4 = dstv[c, pl.ds(g * 16, 16)] * 4
                dx = plsc.load_gather(posv, [s4]) - \
                    plsc.load_gather(posv, [d4])
                dy = plsc.load_gather(posv, [s4 + 1]) - \
                    plsc.load_gather(posv, [d4 + 1])
                dz = plsc.load_gather(posv, [s4 + 2]) - \
                    plsc.load_gather(posv, [d4 + 2])
                dsq = dx * dx + dy * dy + dz * dz
                plsc.store_scatter(bdd, [rbase], dx)
                plsc.store_scatter(bdd, [rbase + 1], dy)
                plsc.store_scatter(bdd, [rbase + 2], dz)
                plsc.store_scatter(bdd, [rbase + 3], dsq)

        def write_out(c, bhs, bhd, bdd):
            row = base + c * CH
            w1 = pltpu.async_copy(bhs, hsg_o.at[pl.ds(row, CH)], wsem)
            w2 = pltpu.async_copy(bhd, hdg_o.at[pl.ds(row, CH)], wsem)
            w3 = pltpu.async_copy(bdd, dd_o.at[pl.ds(row * 16, CH * 16)],
                                  wsem)
            w1.wait()
            w2.wait()
            w3.wait()

        start_gather(0, bhsA, bhdA, gsemA)

        def body2(k, carry):
            c0 = 2 * k
            c1 = c0 + 1
            start_gather(c1, bhsB, bhdB, gsemB)
            pltpu.make_async_copy(hs_r.at[srcv.at[c0]], bhsA, gsemA).wait()
            pltpu.make_async_copy(hd_r.at[dstv.at[c0]], bhdA, gsemA).wait()
            compute_dd(c0, bddA)
            write_out(c0, bhsA, bhdA, bddA)

            @pl.when(k < IW // 2 - 1)
            def _():
                start_gather(c0 + 2, bhsA, bhdA, gsemA)

            pltpu.make_async_copy(hs_r.at[srcv.at[c1]], bhsB, gsemB).wait()
            pltpu.make_async_copy(hd_r.at[dstv.at[c1]], bhdB, gsemB).wait()
            compute_dd(c1, bddB)
            write_out(c1, bhsB, bhdB, bddB)
            return carry

        lax.fori_loop(0, IW // 2, body2, 0)

    return gk(hs, hd, pos4.reshape(NP * 4), src2d, dst2d,
              z16.reshape(CH * 16))


def _sc_scatter_m(m, dst2d, zrt):
    mesh = plsc.VectorSubcoreMesh(core_axis_name="c", subcore_axis_name="s")

    @functools.partial(
        pl.kernel,
        out_type=(jax.ShapeDtypeStruct((NP, H), _f32),
                  jax.ShapeDtypeStruct((NP, H), _f32)),
        mesh=mesh,
        compiler_params=pltpu.CompilerParams(needs_layout_passes=False),
        scratch_types=[
            pltpu.VMEM((IW, CH), _i32),
            pltpu.VMEM((CH, H), _f32),
            pltpu.VMEM((CH, H), _f32),
            pltpu.VMEM_SHARED((NP, H), _f32),
            pltpu.SemaphoreType.DMA,
            pltpu.SemaphoreType.DMA,
        ],
    )
    def sk(m_r, dst_r, zrt_r, agg0_o, agg1_o,
           dstv, bmA, bmB, aggs, rsemA, rsemB):
        cid = lax.axis_index("c")
        sid = lax.axis_index("s")
        wid = sid * 2 + cid
        pltpu.sync_copy(zrt_r, aggs.at[pl.ds(sid * RT, RT)])
        plsc.subcore_barrier()
        pltpu.sync_copy(dst_r.at[pl.ds(wid * IW, IW)], dstv)
        base = wid * EW

        def chunk(c):
            return m_r.at[pl.ds(base + c * CH, CH)]

        pltpu.async_copy(chunk(0), bmA, rsemA)

        def body2(k, carry):
            c0 = 2 * k
            c1 = c0 + 1
            pltpu.async_copy(chunk(c1), bmB, rsemB)
            pltpu.make_async_copy(chunk(c0), bmA, rsemA).wait()
            pltpu.sync_copy(bmA, aggs.at[dstv.at[c0]], add=True)

            @pl.when(k < IW // 2 - 1)
            def _():
                pltpu.async_copy(chunk(c0 + 2), bmA, rsemA)

            pltpu.make_async_copy(chunk(c1), bmB, rsemB).wait()
            pltpu.sync_copy(bmB, aggs.at[dstv.at[c1]], add=True)
            return carry

        lax.fori_loop(0, IW // 2, body2, 0)
        plsc.subcore_barrier()
        rows = pl.ds(sid * RT, RT)

        @pl.when(cid == 0)
        def _():
            pltpu.sync_copy(aggs.at[rows], agg0_o.at[rows])

        @pl.when(cid == 1)
        def _():
            pltpu.sync_copy(aggs.at[rows], agg1_o.at[rows])

    return sk(m, dst2d, zrt)


def _sc_scatter_c(cdx, dst8_2d, zrt):
    mesh = plsc.VectorSubcoreMesh(core_axis_name="c", subcore_axis_name="s")
    CT = N8 // 16

    @functools.partial(
        pl.kernel,
        out_type=(jax.ShapeDtypeStruct((N8, H), _f32),
                  jax.ShapeDtypeStruct((N8, H), _f32)),
        mesh=mesh,
        compiler_params=pltpu.CompilerParams(needs_layout_passes=False),
        scratch_types=[
            pltpu.VMEM((IW, CH), _i32),
            pltpu.VMEM((CH, H), _f32),
            pltpu.VMEM((CH, H), _f32),
            pltpu.VMEM_SHARED((N8, H), _f32),
            pltpu.SemaphoreType.DMA,
            pltpu.SemaphoreType.DMA,
        ],
    )
    def sk(cdx_r, dst8_r, zrt_r, cp0_o, cp1_o,
           dst8v, bcA, bcB, cds, rsemA, rsemB):
        cid = lax.axis_index("c")
        sid = lax.axis_index("s")
        wid = sid * 2 + cid
        pltpu.sync_copy(zrt_r.at[pl.ds(0, CT)], cds.at[pl.ds(sid * CT, CT)])
        plsc.subcore_barrier()
        pltpu.sync_copy(dst8_r.at[pl.ds(wid * IW, IW)], dst8v)
        base = wid * EW

        def chunk(c):
            return cdx_r.at[pl.ds(base + c * CH, CH)]

        pltpu.async_copy(chunk(0), bcA, rsemA)

        def body2(k, carry):
            c0 = 2 * k
            c1 = c0 + 1
            pltpu.async_copy(chunk(c1), bcB, rsemB)
            pltpu.make_async_copy(chunk(c0), bcA, rsemA).wait()
            pltpu.sync_copy(bcA, cds.at[dst8v.at[c0]], add=True)

            @pl.when(k < IW // 2 - 1)
            def _():
                pltpu.async_copy(chunk(c0 + 2), bcA, rsemA)

            pltpu.make_async_copy(chunk(c1), bcB, rsemB).wait()
            pltpu.sync_copy(bcB, cds.at[dst8v.at[c1]], add=True)
            return carry

        lax.fori_loop(0, IW // 2, body2, 0)
        plsc.subcore_barrier()
        rows = pl.ds(sid * CT, CT)

        @pl.when(cid == 0)
        def _():
            pltpu.sync_copy(cds.at[rows], cp0_o.at[rows])

        @pl.when(cid == 1)
        def _():
            pltpu.sync_copy(cds.at[rows], cp1_o.at[rows])

    return sk(cdx, dst8_2d, zrt)


def _row_spec():
    return pl.BlockSpec((1, H), lambda i: (0, 0))


def _wspec():
    return pl.BlockSpec((H, H), lambda i: (0, 0))


def _edge_tc(hsg, hdg, dd, dm8, w1t, mb1, mw2t, mb2,
             cw1t, cb1, cw2t, cb2, with_coord):
    grid = (EP // BE,)

    def msg_matmul(hsg_r, hdg_r, dd_r, w1_r, mb1_r, mw2_r, mb2_r):
        ddb = dd_r[...]
        dsq = ddb[:, 3:4]
        lane8 = lax.broadcasted_iota(_i32, (BE, 8), 1)
        dsq8 = jnp.where(lane8 == 0, dsq, 0.0)
        msg = jnp.concatenate([hsg_r[...], hdg_r[...], dsq8], axis=1)
        m1 = _silu(_dot(msg, w1_r[...], preferred_element_type=_f32)
                   + mb1_r[...])
        m = _silu(_dot(m1, mw2_r[...], preferred_element_type=_f32)
                  + mb2_r[...])
        return ddb, m

    def body_coord(hsg_r, hdg_r, dd_r, dm8_r, w1_r, mb1_r, mw2_r, mb2_r,
                   cw1_r, cb1_r, cw2_r, cb2_r, m_o, cdx_o):
        ddb, m = msg_matmul(hsg_r, hdg_r, dd_r, w1_r, mb1_r, mw2_r, mb2_r)
        m_o[...] = m
        u = _silu(_dot(m, cw1_r[...], preferred_element_type=_f32)
                  + cb1_r[...])
        cw8 = _dot(u, cw2_r[...], preferred_element_type=_f32)
        cwv = cw8[:, 0:1] + cb2_r[...]
        cwv = jnp.clip(cwv, -10.0, 10.0)
        cd16 = ddb * cwv
        tiled = jnp.concatenate([cd16] * 8, axis=1)
        lane = lax.broadcasted_iota(_i32, (BE, H), 1)
        keep = (lane // 16 == dm8_r[...]) & (lane % 16 < 3)
        cdx_o[...] = jnp.where(keep, tiled, 0.0)

    def body_plain(hsg_r, hdg_r, dd_r, w1_r, mb1_r, mw2_r, mb2_r, m_o):
        _, m = msg_matmul(hsg_r, hdg_r, dd_r, w1_r, mb1_r, mw2_r, mb2_r)
        m_o[...] = m

    e_spec = pl.BlockSpec((BE, H), lambda i: (i, 0))
    d_spec = pl.BlockSpec((BE, 16), lambda i: (i, 0))
    w1_spec = pl.BlockSpec((264, H), lambda i: (0, 0))
    if with_coord:
        in_specs = [e_spec, e_spec, d_spec,
                    pl.BlockSpec((BE, 1), lambda i: (i, 0)),
                    w1_spec, _row_spec(), _wspec(), _row_spec(),
                    _wspec(), _row_spec(),
                    pl.BlockSpec((H, 8), lambda i: (0, 0)),
                    pl.BlockSpec((1, 1), lambda i: (0, 0))]
        return pl.pallas_call(
            body_coord,
            grid=grid,
            in_specs=in_specs,
            out_specs=(e_spec, e_spec),
            out_shape=(jax.ShapeDtypeStruct((EP, H), _f32),
                       jax.ShapeDtypeStruct((EP, H), _f32)),
        )(hsg, hdg, dd, dm8, w1t, mb1, mw2t, mb2, cw1t, cb1, cw2t, cb2)
    in_specs = [e_spec, e_spec, d_spec,
                w1_spec, _row_spec(), _wspec(), _row_spec()]
    return pl.pallas_call(
        body_plain,
        grid=grid,
        in_specs=in_specs,
        out_specs=e_spec,
        out_shape=jax.ShapeDtypeStruct((EP, H), _f32),
    )(hsg, hdg, dd, w1t, mb1, mw2t, mb2)


def _node0_tc(nt144, w_in, b_in):
    grid = (NP // BN,)

    def body(nt_r, w_r, b_r, h_o):
        h_o[...] = _dot(nt_r[...], w_r[...], preferred_element_type=_f32) \
            + b_r[...]

    n_spec = pl.BlockSpec((BN, H), lambda i: (i, 0))
    return pl.pallas_call(
        body,
        grid=grid,
        in_specs=[pl.BlockSpec((BN, 144), lambda i: (i, 0)),
                  pl.BlockSpec((144, H), lambda i: (0, 0)),
                  _row_spec()],
        out_specs=n_spec,
        out_shape=jax.ShapeDtypeStruct((NP, H), _f32),
    )(nt144, w_in, b_in)


def _node_mid_tc(h, pos4, agg0, agg1, cda0, cda1, nw1t, nb1, nw2t, nb2):
    grid = (NP // BN,)

    def body(h_r, pos_r, a0_r, a1_r, c0_r, c1_r,
             w1_r, b1_r, w2_r, b2_r, h_o, pos_o):
        agg = a0_r[...] + a1_r[...]
        hn = jnp.concatenate([h_r[...], agg], axis=1)
        act = _silu(_dot(hn, w1_r[...], preferred_element_type=_f32)
                    + b1_r[...])
        h_o[...] = h_r[...] + _dot(act, w2_r[...],
                                   preferred_element_type=_f32) + b2_r[...]
        pos_o[...] = pos_r[...] + c0_r[...][:, :4] + c1_r[...][:, :4]

    n_spec = pl.BlockSpec((BN, H), lambda i: (i, 0))
    p_spec = pl.BlockSpec((BN, 4), lambda i: (i, 0))
    c_spec = pl.BlockSpec((BN, 16), lambda i: (i, 0))
    return pl.pallas_call(
        body,
        grid=grid,
        in_specs=[n_spec, p_spec, n_spec, n_spec, c_spec, c_spec,
                  pl.BlockSpec((2 * H, H), lambda i: (0, 0)), _row_spec(),
                  _wspec(), _row_spec()],
        out_specs=(n_spec, p_spec),
        out_shape=(jax.ShapeDtypeStruct((NP, H), _f32),
                   jax.ShapeDtypeStruct((NP, 4), _f32)),
    )(h, pos4, agg0, agg1, cda0, cda1, nw1t, nb1, nw2t, nb2)


def _node_fin_tc(h, agg0, agg1, nw1t, nb1, nw2t, nb2, cwt, cb, awt, ab):
    grid = (NP // BN,)

    def body(h_r, a0_r, a1_r, w1_r, b1_r, w2_r, b2_r,
             cw_r, cb_r, aw_r, ab_r, co_o, ao_o):
        agg = a0_r[...] + a1_r[...]
        hn = jnp.concatenate([h_r[...], agg], axis=1)
        act = _silu(_dot(hn, w1_r[...], preferred_element_type=_f32)
                    + b1_r[...])
        hf = h_r[...] + _dot(act, w2_r[...],
                             preferred_element_type=_f32) + b2_r[...]
        co_o[...] = _dot(hf, cw_r[...], preferred_element_type=_f32) \
            + cb_r[...]
        ao_o[...] = _dot(hf, aw_r[...], preferred_element_type=_f32) \
            + ab_r[...]

    n_spec = pl.BlockSpec((BN, H), lambda i: (i, 0))
    o_spec = pl.BlockSpec((BN, 16), lambda i: (i, 0))
    sw_spec = pl.BlockSpec((H, 16), lambda i: (0, 0))
    sr_spec = pl.BlockSpec((1, 16), lambda i: (0, 0))
    return pl.pallas_call(
        body,
        grid=grid,
        in_specs=[n_spec, n_spec, n_spec,
                  pl.BlockSpec((2 * H, H), lambda i: (0, 0)), _row_spec(),
                  _wspec(), _row_spec(),
                  sw_spec, sr_spec, sw_spec, sr_spec],
        out_specs=(o_spec, o_spec),
        out_shape=(jax.ShapeDtypeStruct((NP, 16), _f32),
                   jax.ShapeDtypeStruct((NP, 16), _f32)),
    )(h, agg0, agg1, nw1t, nb1, nw2t, nb2, cwt, cb, awt, ab)


def kernel(noisy_types, noisy_pos, edge_index, t, params):
    p = params
    tt = t.reshape(1, 1).astype(_f32) / T
    te = _silu(tt @ p['time_w1'].T + p['time_b1'])
    te = te @ p['time_w2'].T + p['time_b2']

    nt144 = jnp.zeros((NP, 144), _f32)
    nt144 = nt144.at[:N, :NT].set(noisy_types)
    nt144 = nt144.at[:, NT:NT + H].set(jnp.broadcast_to(te, (NP, H)))
    w_in = jnp.zeros((144, H), _f32).at[:NT + H, :].set(p['in_w'].T)
    b_in = p['in_b'].reshape(1, H)

    pos = jnp.zeros((NP, 4), _f32).at[:N, :3].set(noisy_pos)

    src = edge_index[0].astype(_i32)
    dst = edge_index[1].astype(_i32)
    src2d = jnp.zeros((EP,), _i32).at[:E].set(src).reshape(EP // CH, CH)
    dst_p = jnp.full((EP,), N, _i32).at[:E].set(dst)
    dst2d = dst_p.reshape(EP // CH, CH)
    dst8_2d = (dst_p >> 3).reshape(EP // CH, CH)
    dm8 = (dst_p & 7).reshape(EP, 1)

    zrt = jnp.zeros((RT, H), _f32)
    z16 = jnp.zeros((CH, 16), _f32)

    layers = p['layers']
    h = _node0_tc(nt144, w_in, b_in)

    for li in range(4):
        lp = layers[li]
        w1t = jnp.zeros((264, H), _f32).at[:2 * H + 1, :].set(lp['mw1'].T)
        mb1 = lp['mb1'].reshape(1, H)
        mw2t = lp['mw2'].T
        mb2 = lp['mb2'].reshape(1, H)
        nw1t = lp['nw1'].T
        nb1 = lp['nb1'].reshape(1, H)
        nw2t = lp['nw2'].T
        nb2 = lp['nb2'].reshape(1, H)

        hsg, hdg, ddf = _sc_gather(h, h, pos, src2d, dst2d, z16)
        dd = ddf.reshape(EP, 16)

        if li < 3:
            cw1t = lp['cw1'].T
            cb1 = lp['cb1'].reshape(1, H)
            cw2t = jnp.zeros((H, 8), _f32).at[:, 0].set(lp['cw2'][0])
            cb2 = lp['cb2'].reshape(1, 1)
            m, cdx = _edge_tc(hsg, hdg, dd, dm8, w1t, mb1, mw2t, mb2,
                              cw1t, cb1, cw2t, cb2, with_coord=True)
            agg0, agg1 = _sc_scatter_m(m, dst2d, zrt)
            cp0, cp1 = _sc_scatter_c(cdx, dst8_2d, zrt)
            cda0 = cp0.reshape(NP, 16)
            cda1 = cp1.reshape(NP, 16)
            h, pos = _node_mid_tc(h, pos, agg0, agg1, cda0, cda1,
                                  nw1t, nb1, nw2t, nb2)
        else:
            m = _edge_tc(hsg, hdg, dd, None, w1t, mb1, mw2t, mb2,
                         None, None, None, None, with_coord=False)
            agg0, agg1 = _sc_scatter_m(m, dst2d, zrt)
            cwt = jnp.zeros((H, 16), _f32).at[:, :3].set(p['coord_w'].T)
            cb = jnp.zeros((1, 16), _f32).at[0, :3].set(p['coord_b'])
            awt = jnp.zeros((H, 16), _f32).at[:, :NT].set(p['atom_w'].T)
            ab = jnp.zeros((1, 16), _f32).at[0, :NT].set(p['atom_b'])
            co, ao = _node_fin_tc(h, agg0, agg1, nw1t, nb1,
                                  nw2t, nb2, cwt, cb, awt, ab)

    return (co[:N, :3], ao[:N, :NT])
